# baseline (speedup 1.0000x reference)
import functools

import jax
import jax.numpy as jnp
from jax import lax
from jax.experimental import pallas as pl
from jax.experimental.pallas import tpu as pltpu
from jax.experimental.pallas import tpu_sc as plsc

N = 10000
D = 128
NP = 10112
E = 320000
NC, NS, L = 2, 16, 16
NW = NC * NS
KE = 64
CH = 160
EPT = CH * KE
EPAD = EPT * NW
STRIPE = NP // NS
NBUF = 4
NIDX = 8


@functools.cache
def _mesh():
    return plsc.VectorSubcoreMesh(
        core_axis_name="c", subcore_axis_name="s", num_cores=NC, num_subcores=NS)


def _deg_body(colp_hbm, out_hbm, col4, ones_v, zbuf, si0, si1, si2, si3,
              ss0, ss1, ss2, ss3, acc):
    si = (si0, si1, si2, si3)
    ss = (ss0, ss1, ss2, ss3)
    cid = lax.axis_index("c")
    sid = lax.axis_index("s")
    wid = cid * NS + sid
    base = wid * EPT
    for i in range(STRIPE // L + 1):
        zbuf[pl.ds(i * L, L)] = jnp.zeros((L,), jnp.float32)
    for i in range(KE // L):
        ones_v[pl.ds(i * L, L)] = jnp.full((L,), 1.0, jnp.float32)
    pltpu.sync_copy(zbuf.at[pl.ds(0, STRIPE)], acc.at[pl.ds(sid * STRIPE, STRIPE)])
    plsc.subcore_barrier()

    def idx_src(q):
        return colp_hbm.at[pl.ds(pl.multiple_of(base + q * KE, KE), KE)]

    pltpu.async_copy(idx_src(0), col4.at[0], si[0])
    pltpu.async_copy(idx_src(1), col4.at[1], si[1])

    def body(i, carry):
        for b in range(4):
            v = i * 4 + b
            b2 = (b + 2) % 4
            pltpu.make_async_copy(idx_src(v), col4.at[b], si[b]).wait()
            pltpu.async_copy(ones_v, acc.at[col4.at[b]], ss[b], add=True)
            w = v + 2

            @pl.when(jnp.logical_and(w < CH, w >= 4))
            def _():
                pltpu.make_async_copy(ones_v, acc.at[col4.at[b2]], ss[b2]).wait()

            @pl.when(w < CH)
            def _():
                pltpu.async_copy(idx_src(w), col4.at[b2], si[b2])
        return carry

    lax.fori_loop(0, CH // 4, body, 0)
    for b in range(4):
        pltpu.make_async_copy(ones_v, acc.at[col4.at[b]], ss[b]).wait()
    plsc.subcore_barrier()
    off = pl.multiple_of(cid * NP + sid * STRIPE, 8)
    pltpu.sync_copy(acc.at[pl.ds(sid * STRIPE, STRIPE)], zbuf.at[pl.ds(0, STRIPE)])
    pltpu.sync_copy(zbuf.at[pl.ds(0, STRIPE)], out_hbm.at[pl.ds(off, STRIPE)])


@functools.cache
def _sc_deg_kernel():
    return pl.kernel(
        _deg_body,
        out_type=jax.ShapeDtypeStruct((NC * NP,), jnp.float32),
        mesh=_mesh(),
        scratch_types=[
            pltpu.VMEM((4, KE), jnp.int32),
            pltpu.VMEM((KE,), jnp.float32),
            pltpu.VMEM((STRIPE // L * L + L,), jnp.float32),
        ] + [pltpu.SemaphoreType.DMA] * 8 + [
            pltpu.VMEM_SHARED((NP,), jnp.float32),
        ],
    )


def _sc_deg(colp):
    return _sc_deg_kernel()(colp).reshape(NC, NP)


def _prop_body(u_hbm, colp_hbm, rowp_hbm, out_hbm, col8, row8,
               r0, r1, r2, r3, sg0, sg1, sg2, sg3, ss0, ss1, ss2, ss3,
               si0, si1, si2, si3, si4, si5, si6, si7, acc):
    rows = (r0, r1, r2, r3)
    sg = (sg0, sg1, sg2, sg3)
    ss = (ss0, ss1, ss2, ss3)
    si = (si0, si1, si2, si3, si4, si5, si6, si7)
    cid = lax.axis_index("c")
    sid = lax.axis_index("s")
    wid = cid * NS + sid
    base = wid * EPT
    for o in range(0, STRIPE, KE):
        sz = min(KE, STRIPE - o)
        pltpu.sync_copy(u_hbm.at[pl.ds(sid * STRIPE + o, sz)], r0.at[pl.ds(0, sz)])
        pltpu.sync_copy(r0.at[pl.ds(0, sz)], acc.at[pl.ds(sid * STRIPE + o, sz)])
    plsc.subcore_barrier()

    def cidx_src(q):
        return colp_hbm.at[pl.ds(pl.multiple_of(base + q * KE, KE), KE)]

    def ridx_src(q):
        return rowp_hbm.at[pl.ds(pl.multiple_of(base + q * KE, KE), KE)]

    for q in range(6):
        pltpu.async_copy(cidx_src(q), col8.at[q], si[q])
        pltpu.async_copy(ridx_src(q), row8.at[q], si[q])
    for v in range(2):
        pltpu.make_async_copy(cidx_src(v), col8.at[v], si[v]).wait()
        pltpu.make_async_copy(ridx_src(v), row8.at[v], si[v]).wait()
        pltpu.async_copy(u_hbm.at[col8.at[v]], rows[v], sg[v])

    def body(i, carry):
        for k in range(8):
            v = i * 8 + k
            b = k % 4
            b2 = (b + 2) % 4
            c2 = (k + 2) % 8
            c6 = (k + 6) % 8
            pltpu.make_async_copy(u_hbm.at[col8.at[k]], rows[b], sg[b]).wait()
            pltpu.async_copy(rows[b], acc.at[row8.at[k]], ss[b], add=True)
            w = v + 2

            @pl.when(jnp.logical_and(w < CH, w >= 4))
            def _():
                pltpu.make_async_copy(rows[b2], acc.at[row8.at[c2]], ss[b2]).wait()

            @pl.when(w < CH)
            def _():
                pltpu.make_async_copy(cidx_src(w), col8.at[c2], si[c2]).wait()
                pltpu.make_async_copy(ridx_src(w), row8.at[c2], si[c2]).wait()
                pltpu.async_copy(u_hbm.at[col8.at[c2]], rows[b2], sg[b2])

            q = v + 6

            @pl.when(q < CH)
            def _():
                pltpu.async_copy(cidx_src(q), col8.at[c6], si[c6])
                pltpu.async_copy(ridx_src(q), row8.at[c6], si[c6])
        return carry

    lax.fori_loop(0, CH // 8, body, 0)
    for j in range(4):
        b = (CH - 4 + j) % 4
        c = (CH - 4 + j) % 8
        pltpu.make_async_copy(rows[b], acc.at[row8.at[c]], ss[b]).wait()
    plsc.subcore_barrier()
    for o in range(0, STRIPE, KE):
        sz = min(KE, STRIPE - o)
        pltpu.sync_copy(acc.at[pl.ds(sid * STRIPE + o, sz)], r0.at[pl.ds(0, sz)])
        pltpu.sync_copy(r0.at[pl.ds(0, sz)],
                        out_hbm.at[cid, pl.ds(sid * STRIPE + o, sz)])


@functools.cache
def _sc_prop_kernel():
    return pl.kernel(
        _prop_body,
        out_type=jax.ShapeDtypeStruct((NC, NP, D), jnp.float32),
        mesh=_mesh(),
        scratch_types=[
            pltpu.VMEM((NIDX, KE), jnp.int32),
            pltpu.VMEM((NIDX, KE), jnp.int32),
            pltpu.VMEM((KE, D), jnp.float32),
            pltpu.VMEM((KE, D), jnp.float32),
            pltpu.VMEM((KE, D), jnp.float32),
            pltpu.VMEM((KE, D), jnp.float32),
        ] + [pltpu.SemaphoreType.DMA] * 16 + [
            pltpu.VMEM_SHARED((NP, D), jnp.float32),
        ],
    )


def _sc_prop(u, colp, rowp):
    return _sc_prop_kernel()(u, colp, rowp)


def kernel(x, edge_index, W0_0, b0_0, W0_1, b0_1, W0_2, b0_2, bn_g, bn_b,
           W1_0, b1_0, W1_1, b1_1, W1_2, b1_2, Wf, bf):
    row = edge_index[0]
    col = edge_index[1]
    pad_idx = jnp.full((EPAD - E,), N, jnp.int32)
    rowp = jnp.concatenate([row, pad_idx])
    colp = jnp.concatenate([col, pad_idx])

    degp = _sc_deg(colp)
    deg = degp[0] + degp[1] + 1.0
    dis = lax.rsqrt(deg)[:, None]

    xp = jnp.pad(x, ((0, NP - N), (0, 0)))

    def prop(tp):
        u = dis * tp
        p = _sc_prop(u, colp, rowp)
        return dis * (p[0] + p[1] - u)

    def mixhop(hp, Ws, bs):
        outs = []
        for j, (W, b) in enumerate(zip(Ws, bs)):
            hj = hp @ W.T + b
            for _ in range(j):
                hj = prop(hj)
            outs.append(hj)
        return jnp.concatenate(outs, axis=1)

    h = mixhop(xp, [W0_0, W0_1, W0_2], [b0_0, b0_1, b0_2])
    hn = h[:N]
    mean = hn.mean(axis=0)
    var = hn.var(axis=0)
    hn = (hn - mean) / jnp.sqrt(var + 1e-5) * bn_g + bn_b
    hn = jax.nn.relu(hn)
    hp = jnp.pad(hn, ((0, NP - N), (0, 0)))
    h2 = mixhop(hp, [W1_0, W1_1, W1_2], [b1_0, b1_1, b1_2])
    return (h2 @ Wf.T + bf)[:N]
